# baseline (speedup 1.0000x reference)
import functools

import jax
import jax.numpy as jnp
from jax import lax
from jax.experimental import pallas as pl
from jax.experimental.pallas import tpu as pltpu
from jax.experimental.pallas import tpu_sc as plsc

NF = 384
B = 32
HW = 1024
CH = 128
NCH = NF // CH

NWORK = 4
RPW = B // NWORK


def _make_gather():
    mesh = plsc.VectorSubcoreMesh(core_axis_name="c", subcore_axis_name="s")

    @functools.partial(
        pl.kernel,
        mesh=mesh,
        out_type=jax.ShapeDtypeStruct((B, 2 * NF), jnp.float32),
        scratch_types=[
            pltpu.VMEM((RPW,), jnp.int32),
            pltpu.VMEM((RPW, 2 * NF), jnp.float32),
            pltpu.SemaphoreType.DMA,
        ],
    )
    def gather(table_hbm, idx_hbm, out_hbm, idx_v, rows_v, sem):
        wid = lax.axis_index("s") * 2 + lax.axis_index("c")

        @pl.when(wid < NWORK)
        def _():
            base = wid * RPW
            pltpu.sync_copy(idx_hbm.at[pl.ds(base, RPW)], idx_v)
            pltpu.async_copy(table_hbm.at[idx_v], rows_v, sem).wait()
            pltpu.sync_copy(rows_v, out_hbm.at[pl.ds(base, RPW)])

    return gather


_gather = _make_gather()


def _affine_body(e_ref, x_ref, o_ref):
    j = pl.program_id(1)
    off = pl.multiple_of(j * CH, 8)
    g = e_ref[0, pl.ds(off, CH)]
    b = e_ref[0, pl.ds(NF + off, CH)]
    o_ref[0] = x_ref[0] * g + b


def kernel(x, y, embed_weight):
    y32 = y.astype(jnp.int32)
    e = _gather(embed_weight, y32)
    e4 = e.reshape(B, 2 * NF, 1, 1)
    H = x.shape[2]
    W = x.shape[3]
    out = pl.pallas_call(
        _affine_body,
        grid=(B, NCH),
        in_specs=[
            pl.BlockSpec((1, 2 * NF, 1, 1), lambda bi, j: (bi, 0, 0, 0)),
            pl.BlockSpec((1, CH, H, W), lambda bi, j: (bi, j, 0, 0)),
        ],
        out_specs=pl.BlockSpec((1, CH, H, W), lambda bi, j: (bi, j, 0, 0)),
        out_shape=jax.ShapeDtypeStruct(x.shape, jnp.float32),
    )(e4, x)
    return out


# speedup vs baseline: 2.6667x; 2.6667x over previous
import functools

import jax
import jax.numpy as jnp
from jax import lax
from jax.experimental import pallas as pl
from jax.experimental.pallas import tpu as pltpu
from jax.experimental.pallas import tpu_sc as plsc

NF = 384
B = 32
HW = 1024
CH = 128
NCH = NF // CH

NWORK = 4
RPW = B // NWORK


def _make_gather():
    mesh = plsc.VectorSubcoreMesh(core_axis_name="c", subcore_axis_name="s")

    @functools.partial(
        pl.kernel,
        mesh=mesh,
        out_type=jax.ShapeDtypeStruct((B, 2 * NF), jnp.float32),
        scratch_types=[
            pltpu.VMEM((RPW,), jnp.int32),
            pltpu.VMEM((RPW, 2 * NF), jnp.float32),
            pltpu.SemaphoreType.DMA,
        ],
    )
    def gather(table_hbm, idx_hbm, out_hbm, idx_v, rows_v, sem):
        wid = lax.axis_index("s") * 2 + lax.axis_index("c")

        @pl.when(wid < NWORK)
        def _():
            base = wid * RPW
            pltpu.sync_copy(idx_hbm.at[pl.ds(base, RPW)], idx_v)
            pltpu.async_copy(table_hbm.at[idx_v], rows_v, sem).wait()
            pltpu.sync_copy(rows_v, out_hbm.at[pl.ds(base, RPW)])

    return gather


_gather = _make_gather()


def _affine_body(e_ref, x_ref, o_ref):
    bi = pl.program_id(0)
    j = pl.program_id(1)
    off = pl.multiple_of(j * CH, 128)
    g = e_ref[bi, 0, pl.ds(off, CH)].reshape(CH, 1)
    b = e_ref[bi, 0, pl.ds(NF + off, CH)].reshape(CH, 1)
    o_ref[0] = x_ref[0] * g + b


def kernel(x, y, embed_weight):
    y32 = y.astype(jnp.int32)
    e = _gather(embed_weight, y32)
    xr = x.reshape(B, NF, HW)
    out = pl.pallas_call(
        _affine_body,
        grid=(B, NCH),
        in_specs=[
            pl.BlockSpec((B, 1, 2 * NF), lambda bi, j: (0, 0, 0)),
            pl.BlockSpec((1, CH, HW), lambda bi, j: (bi, j, 0)),
        ],
        out_specs=pl.BlockSpec((1, CH, HW), lambda bi, j: (bi, j, 0)),
        out_shape=jax.ShapeDtypeStruct((B, NF, HW), jnp.float32),
    )(e.reshape(B, 1, 2 * NF), xr)
    return out.reshape(x.shape)
